# baseline (speedup 1.0000x reference)
import functools

import jax
import jax.numpy as jnp
from jax import lax
from jax.experimental import pallas as pl
from jax.experimental.pallas import tpu as pltpu
from jax.experimental.pallas import tpu_sc as plsc

B = 64
P = 8192
D = 128
K = 64


def _sim_body(lf_ref, proto_ref, nrm_ref, sim_ref):
    x = lf_ref[0]
    p = proto_ref[0]
    pn = p / (jnp.sqrt(jnp.sum(p * p)) + 1e-8)
    nrc = jnp.transpose(nrm_ref[0], (1, 0))
    ln = x / (nrc + 1e-8)
    lnb = ln.astype(jnp.bfloat16).astype(jnp.float32)
    pnb = pn.astype(jnp.bfloat16).astype(jnp.float32)
    sim_ref[0, 0] = jnp.sum(lnb * pnb, axis=1)


def _sim(local_features, prototypes):
    nrm = jnp.linalg.norm(local_features, axis=-1)[:, None, :]
    out = pl.pallas_call(
        _sim_body,
        grid=(B,),
        in_specs=[
            pl.BlockSpec((1, P, D), lambda b: (b, 0, 0)),
            pl.BlockSpec((1, 1, D), lambda b: (b, 0, 0)),
            pl.BlockSpec((1, 1, P), lambda b: (b, 0, 0)),
        ],
        out_specs=pl.BlockSpec((1, 1, P), lambda b: (b, 0, 0)),
        out_shape=jax.ShapeDtypeStruct((B, 1, P), jnp.float32),
    )(local_features, prototypes, nrm)
    return out.reshape(B, P)


def _topk_body(sim_ref, ti_ref, bi_ref, st_ref, sb_ref):
    iota = lax.broadcasted_iota(jnp.int32, (B, P), 1)
    kio = lax.broadcasted_iota(jnp.int32, (B, K), 1)
    inf = jnp.float32(jnp.inf)
    s0 = sim_ref[...]
    st_ref[...] = s0
    sb_ref[...] = s0
    vt0 = jnp.max(s0, axis=1, keepdims=True)
    vb0 = jnp.min(s0, axis=1, keepdims=True)

    def step(k, carry):
        ti, bi, vt, vb = carry
        st = st_ref[...]
        sb = sb_ref[...]
        it = jnp.min(jnp.where(st == vt, iota, P), axis=1, keepdims=True)
        ib = jnp.min(jnp.where(sb == vb, iota, P), axis=1, keepdims=True)
        st2 = jnp.where(iota == it, -inf, st)
        sb2 = jnp.where(iota == ib, inf, sb)
        st_ref[...] = st2
        sb_ref[...] = sb2
        vt2 = jnp.max(st2, axis=1, keepdims=True)
        vb2 = jnp.min(sb2, axis=1, keepdims=True)
        sel = kio == k
        ti = jnp.where(sel, it, ti)
        bi = jnp.where(sel, ib, bi)
        return ti, bi, vt2, vb2

    zero = jnp.zeros((B, K), jnp.int32)
    ti, bi, _, _ = lax.fori_loop(0, K, step, (zero, zero, vt0, vb0))
    ti_ref[...] = ti
    bi_ref[...] = bi


def _topk(sim):
    return pl.pallas_call(
        _topk_body,
        out_shape=[
            jax.ShapeDtypeStruct((B, K), jnp.int32),
            jax.ShapeDtypeStruct((B, K), jnp.int32),
        ],
        scratch_shapes=[
            pltpu.VMEM((B, P), jnp.float32),
            pltpu.VMEM((B, P), jnp.float32),
        ],
    )(sim)


_NROWS = 2 * B * K


@functools.cache
def _make_sc_gather():
    info = plsc.get_sparse_core_info()
    nw = info.num_cores * info.num_subcores
    bpw = _NROWS // nw
    mesh = plsc.VectorSubcoreMesh(core_axis_name="c", subcore_axis_name="s")

    @functools.partial(
        pl.kernel,
        mesh=mesh,
        out_type=jax.ShapeDtypeStruct((_NROWS, D), jnp.float32),
        scratch_types=[
            pltpu.VMEM((bpw,), jnp.int32),
            pltpu.VMEM((bpw, D), jnp.float32),
            pltpu.SemaphoreType.DMA,
        ],
    )
    def gather(table_hbm, idx_hbm, out_hbm, idx_v, rows_v, sem):
        wid = lax.axis_index("s") * info.num_cores + lax.axis_index("c")
        base = wid * bpw
        pltpu.sync_copy(idx_hbm.at[pl.ds(base, bpw)], idx_v)
        pltpu.async_copy(table_hbm.at[idx_v], rows_v, sem).wait()
        pltpu.sync_copy(rows_v, out_hbm.at[pl.ds(base, bpw)])

    return gather


@jax.jit
def kernel(local_features, prototypes):
    sim = _sim(local_features, prototypes)
    ti, bi = _topk(sim)
    offs = (jnp.arange(B, dtype=jnp.int32) * P)[:, None]
    flat_idx = jnp.concatenate([ti + offs, bi + offs], axis=0).reshape(-1)
    table = local_features.reshape(B * P, D)
    rows = _make_sc_gather()(table, flat_idx).reshape(2, B, K, D)
    return rows[0], rows[1], ti, bi
